# baseline (speedup 1.0000x reference)
import functools

import jax
import jax.numpy as jnp
from jax import lax
from jax.experimental import pallas as pl
from jax.experimental.pallas import tpu as pltpu
from jax.experimental.pallas import tpu_sc as plsc

NUM_EXPERTS = 8
TOP_K = 2
HIDDEN = 768
INTERMEDIATE = 512
TOKENS = 2048
ROWS = TOKENS * TOP_K

NC = 2
NS = 16
NW = NC * NS
TPW = TOKENS // NW
BM = 512
NB = ROWS // BM + NUM_EXPERTS - 1
NBP = 16

_MESH = plsc.VectorSubcoreMesh(core_axis_name="c", subcore_axis_name="s")
_HI = jax.lax.Precision.HIGHEST


def _wid():
    return lax.axis_index("c") * NS + lax.axis_index("s")


def _route_body(lg_ref, a0_ref, a1_ref, w0_ref, w1_ref, carry_ref, offs_ref):
    logits = lg_ref[...]
    iota = lax.broadcasted_iota(jnp.int32, (TOKENS, NUM_EXPERTS), 1)
    big = jnp.int32(NUM_EXPERTS)
    m1 = jnp.max(logits, axis=1, keepdims=True)
    e0 = jnp.min(jnp.where(logits == m1, iota, big), axis=1, keepdims=True)
    masked = jnp.where(iota == e0, -jnp.inf, logits)
    m2 = jnp.max(masked, axis=1, keepdims=True)
    e1 = jnp.min(jnp.where(masked == m2, iota, big), axis=1, keepdims=True)
    w0 = 1.0 / (1.0 + jnp.exp(m2 - m1))
    a0_ref[...] = e0
    a1_ref[...] = e1
    w0_ref[...] = w0
    w1_ref[...] = 1.0 - w0

    oh = (jnp.logical_or(iota == e0, iota == e1)).astype(jnp.float32)
    oh16 = jnp.concatenate([oh, jnp.zeros((TOKENS, 16 - NUM_EXPERTS), jnp.float32)], axis=1)
    wi = lax.broadcasted_iota(jnp.int32, (NW, TOKENS), 0)
    ti = lax.broadcasted_iota(jnp.int32, (NW, TOKENS), 1) // TPW
    seg = (wi == ti).astype(jnp.float32)
    cnt = jnp.dot(seg, oh16, precision=_HI)
    lw = lax.broadcasted_iota(jnp.int32, (NW, NW), 0)
    lwp = lax.broadcasted_iota(jnp.int32, (NW, NW), 1)
    pre = jnp.dot((lwp < lw).astype(jnp.float32), cnt, precision=_HI)
    tot = jnp.sum(cnt, axis=0, keepdims=True)
    le = lax.broadcasted_iota(jnp.int32, (16, 16), 0)
    lep = lax.broadcasted_iota(jnp.int32, (16, 16), 1)
    offs = jnp.dot(tot, (le < lep).astype(jnp.float32), precision=_HI)
    carry_ref[...] = (offs + pre).astype(jnp.int32)
    offs_ref[...] = offs.astype(jnp.int32)


def _route(router_logits):
    return pl.pallas_call(
        _route_body,
        out_shape=(
            jax.ShapeDtypeStruct((TOKENS, 1), jnp.int32),
            jax.ShapeDtypeStruct((TOKENS, 1), jnp.int32),
            jax.ShapeDtypeStruct((TOKENS, 1), jnp.float32),
            jax.ShapeDtypeStruct((TOKENS, 1), jnp.float32),
            jax.ShapeDtypeStruct((NW, 16), jnp.int32),
            jax.ShapeDtypeStruct((1, 16), jnp.int32),
        ),
    )(router_logits)


def _scatter_body(x_hbm, a0_hbm, a1_hbm, carry_hbm, offs_hbm,
                  xs_hbm, p0_hbm, p1_hbm, bg_hbm, br_hbm,
                  a0_v, a1_v, p0_v, p1_v, carr_v, offs_v, mv_v, xt_v,
                  nxt_s, meta_s, sem, semx):
    w = _wid()
    base = w * TPW
    cx = pltpu.async_copy(x_hbm.at[pl.ds(base, TPW)], xt_v, semx)
    pltpu.sync_copy(carry_hbm.at[w], carr_v)
    pltpu.sync_copy(a0_hbm.at[pl.ds(base, TPW)], a0_v)
    pltpu.sync_copy(a1_hbm.at[pl.ds(base, TPW)], a1_v)
    carr = carr_v[...]
    for e in range(NUM_EXPERTS):
        nxt_s[e] = carr[e]
    lanes = lax.iota(jnp.int32, 16)

    @pl.when(w == 0)
    def _():
        pltpu.sync_copy(offs_hbm, offs_v)
        ov = offs_v[...]
        for e in range(NUM_EXPERTS + 1):
            meta_s[e] = ov[e]
        meta_s[16] = 0
        for sl2 in range(NBP):
            meta_s[17 + sl2] = 0
            meta_s[17 + NBP + sl2] = 0
        for m in range(ROWS // BM):
            for e in range(NUM_EXPERTS):
                hit = jnp.logical_and(meta_s[e] < (m + 1) * BM,
                                      meta_s[e + 1] > m * BM)

                @pl.when(hit)
                def _(m=m, e=e):
                    sl = meta_s[16]
                    meta_s[17 + sl] = e
                    meta_s[17 + NBP + sl] = m
                    meta_s[16] = sl + 1
        last = meta_s[16] - 1
        lb = meta_s[17 + last]
        lr = meta_s[17 + NBP + last]
        nv = meta_s[16]
        for h in range(NBP // 16):
            vb = jnp.zeros((16,), jnp.int32)
            vr = jnp.zeros((16,), jnp.int32)
            for l in range(16):
                sl2 = h * 16 + l
                pad = jnp.int32(sl2) >= nv
                vb = jnp.where(lanes == l,
                               jnp.where(pad, lb, meta_s[17 + sl2]), vb)
                vr = jnp.where(lanes == l,
                               jnp.where(pad, lr, meta_s[17 + NBP + sl2]), vr)
            mv_v[pl.ds(h * 16, 16)] = vb
            mv_v[pl.ds(NBP + h * 16, 16)] = vr
        pltpu.sync_copy(mv_v.at[pl.ds(0, NBP)], bg_hbm)
        pltpu.sync_copy(mv_v.at[pl.ds(NBP, NBP)], br_hbm)
    for c4 in range(TPW // 16):
        sl = pl.ds(c4 * 16, 16)
        for a_v_, p_v_ in ((a0_v, p0_v), (a1_v, p1_v)):
            ac = a_v_[sl]
            pos = jnp.zeros((16,), jnp.int32)
            for i in range(16):
                e_s = ac[i]
                p = nxt_s[e_s]
                nxt_s[e_s] = p + 1
                pos = jnp.where(lanes == i, p, pos)
            p_v_[sl] = pos
    cx.wait()
    c0 = pltpu.async_copy(xt_v, xs_hbm.at[p0_v], sem)
    c1 = pltpu.async_copy(xt_v, xs_hbm.at[p1_v], sem)
    pltpu.sync_copy(p0_v, p0_hbm.at[pl.ds(base, TPW)])
    pltpu.sync_copy(p1_v, p1_hbm.at[pl.ds(base, TPW)])
    c0.wait()
    c1.wait()


_scatter = functools.partial(
    pl.kernel,
    out_type=(
        jax.ShapeDtypeStruct((ROWS, HIDDEN), jnp.float32),
        jax.ShapeDtypeStruct((TOKENS,), jnp.int32),
        jax.ShapeDtypeStruct((TOKENS,), jnp.int32),
        jax.ShapeDtypeStruct((NBP,), jnp.int32),
        jax.ShapeDtypeStruct((NBP,), jnp.int32),
    ),
    mesh=_MESH,
    scratch_types=[
        pltpu.VMEM((TPW,), jnp.int32),
        pltpu.VMEM((TPW,), jnp.int32),
        pltpu.VMEM((TPW,), jnp.int32),
        pltpu.VMEM((TPW,), jnp.int32),
        pltpu.VMEM((16,), jnp.int32),
        pltpu.VMEM((16,), jnp.int32),
        pltpu.VMEM((2 * NBP,), jnp.int32),
        pltpu.VMEM((TPW, HIDDEN), jnp.float32),
        pltpu.SMEM((16,), jnp.int32),
        pltpu.SMEM((96,), jnp.int32),
        pltpu.SemaphoreType.DMA,
        pltpu.SemaphoreType.DMA,
    ],
)(_scatter_body)


def _gmm_body(bg_ref, br_ref, offs_ref, xs_ref, gw_ref, uw_ref, dw_ref, out_ref):
    i = pl.program_id(0)
    e = bg_ref[i]
    m = br_ref[i]
    base = m * BM
    ip = jnp.maximum(i - 1, 0)
    prev_m = jnp.where(i > 0, br_ref[ip], -1)
    prev_e = jnp.where(i > 0, bg_ref[ip], -1)
    dup = jnp.logical_and(m == prev_m, e == prev_e)
    first = m != prev_m
    lo = jnp.maximum(offs_ref[e] - base, 0)
    hi = jnp.minimum(offs_ref[e + 1] - base, BM)
    rows = jax.lax.broadcasted_iota(jnp.int32, (BM, 1), 0)

    @pl.when(first)
    def _():
        out_ref[...] = jnp.zeros_like(out_ref)

    @pl.when(jnp.logical_not(dup))
    def _():
        x = xs_ref[...].astype(jnp.bfloat16)
        gw = gw_ref[0].astype(jnp.bfloat16)
        uw = uw_ref[0].astype(jnp.bfloat16)
        dw = dw_ref[0].astype(jnp.bfloat16)
        g = jnp.dot(x, gw, preferred_element_type=jnp.float32)
        u = jnp.dot(x, uw, preferred_element_type=jnp.float32)
        h = ((g * jax.nn.sigmoid(g)) * u).astype(jnp.bfloat16)
        y = jnp.dot(h, dw, preferred_element_type=jnp.float32)
        msk = jnp.logical_and(rows >= lo, rows < hi)
        out_ref[...] += jnp.where(msk, y, 0.0)


def _gmm(bg, br, offs, xs, gw, uw, dw):
    return pl.pallas_call(
        _gmm_body,
        grid_spec=pltpu.PrefetchScalarGridSpec(
            num_scalar_prefetch=3,
            grid=(NB,),
            in_specs=[
                pl.BlockSpec((BM, HIDDEN), lambda i, bg, br, offs: (br[i], 0)),
                pl.BlockSpec((1, HIDDEN, INTERMEDIATE),
                             lambda i, bg, br, offs: (bg[i], 0, 0)),
                pl.BlockSpec((1, HIDDEN, INTERMEDIATE),
                             lambda i, bg, br, offs: (bg[i], 0, 0)),
                pl.BlockSpec((1, INTERMEDIATE, HIDDEN),
                             lambda i, bg, br, offs: (bg[i], 0, 0)),
            ],
            out_specs=pl.BlockSpec((BM, HIDDEN), lambda i, bg, br, offs: (br[i], 0)),
        ),
        out_shape=jax.ShapeDtypeStruct((ROWS, HIDDEN), jnp.float32),
    )(bg, br, offs, xs, gw, uw, dw)


def _combine_body(y_hbm, p0_hbm, p1_hbm, w0_hbm, w1_hbm, out_hbm,
                  p0_v, p1_v, w0_v, w1_v, r0_v, r1_v, o_v, sem):
    w = _wid()
    base = w * TPW
    pltpu.sync_copy(p0_hbm.at[pl.ds(base, TPW)], p0_v)
    pltpu.sync_copy(p1_hbm.at[pl.ds(base, TPW)], p1_v)
    pltpu.sync_copy(w0_hbm.at[pl.ds(base, TPW)], w0_v)
    pltpu.sync_copy(w1_hbm.at[pl.ds(base, TPW)], w1_v)
    HT = TPW // 2
    for hh in range(2):
        g0 = pltpu.async_copy(y_hbm.at[p0_v.at[pl.ds(hh * HT, HT)]], r0_v, sem)
        g1 = pltpu.async_copy(y_hbm.at[p1_v.at[pl.ds(hh * HT, HT)]], r1_v, sem)
        g0.wait()
        g1.wait()
        for g in range(HT // 16):
            w0g = w0_v[pl.ds(hh * HT + g * 16, 16)]
            w1g = w1_v[pl.ds(hh * HT + g * 16, 16)]

            def body(f, _, g=g, w0g=w0g, w1g=w1g):
                for k in range(2):
                    sl = pl.ds(f * 32 + k * 16, 16)
                    for i in range(16):
                        r = g * 16 + i
                        o_v[r, sl] = w0g[i] * r0_v[r, sl] + w1g[i] * r1_v[r, sl]
                return 0

            lax.fori_loop(0, HIDDEN // 32, body, 0)
        pltpu.sync_copy(o_v, out_hbm.at[pl.ds(base + hh * HT, HT)])


_combine = functools.partial(
    pl.kernel,
    out_type=jax.ShapeDtypeStruct((TOKENS, HIDDEN), jnp.float32),
    mesh=_MESH,
    scratch_types=[
        pltpu.VMEM((TPW,), jnp.int32),
        pltpu.VMEM((TPW,), jnp.int32),
        pltpu.VMEM((TPW,), jnp.float32),
        pltpu.VMEM((TPW,), jnp.float32),
        pltpu.VMEM((TPW // 2, HIDDEN), jnp.float32),
        pltpu.VMEM((TPW // 2, HIDDEN), jnp.float32),
        pltpu.VMEM((TPW // 2, HIDDEN), jnp.float32),
        pltpu.SemaphoreType.DMA,
    ],
)(_combine_body)


def kernel(hidden_states, router_logits, gate_w, up_w, down_w):
    a0, a1, w0, w1, carry, offs = _route(router_logits)
    offs_flat = offs.reshape(16)
    xs, p0, p1, bg, br = _scatter(hidden_states, a0.reshape(TOKENS),
                                  a1.reshape(TOKENS), carry, offs_flat)
    ys = _gmm(bg, br, offs_flat, xs, gate_w, up_w, down_w)
    return _combine(ys, p0, p1, w0.reshape(TOKENS), w1.reshape(TOKENS))


# speedup vs baseline: 1.1437x; 1.1437x over previous
import functools

import jax
import jax.numpy as jnp
from jax import lax
from jax.experimental import pallas as pl
from jax.experimental.pallas import tpu as pltpu
from jax.experimental.pallas import tpu_sc as plsc

NUM_EXPERTS = 8
TOP_K = 2
HIDDEN = 768
INTERMEDIATE = 512
TOKENS = 2048
ROWS = TOKENS * TOP_K

NC = 2
NS = 16
NW = NC * NS
TPW = TOKENS // NW
BM = 512
NB = ROWS // BM + NUM_EXPERTS - 1
NBP = 16

_MESH = plsc.VectorSubcoreMesh(core_axis_name="c", subcore_axis_name="s")
_HI = jax.lax.Precision.HIGHEST


def _wid():
    return lax.axis_index("c") * NS + lax.axis_index("s")


def _route_body(lg_ref, a0_ref, a1_ref, w0_ref, w1_ref, carry_ref, offs_ref):
    logits = lg_ref[...]
    iota = lax.broadcasted_iota(jnp.int32, (TOKENS, NUM_EXPERTS), 1)
    big = jnp.int32(NUM_EXPERTS)
    m1 = jnp.max(logits, axis=1, keepdims=True)
    e0 = jnp.min(jnp.where(logits == m1, iota, big), axis=1, keepdims=True)
    masked = jnp.where(iota == e0, -jnp.inf, logits)
    m2 = jnp.max(masked, axis=1, keepdims=True)
    e1 = jnp.min(jnp.where(masked == m2, iota, big), axis=1, keepdims=True)
    w0 = 1.0 / (1.0 + jnp.exp(m2 - m1))
    a0_ref[...] = e0
    a1_ref[...] = e1
    w0_ref[...] = w0
    w1_ref[...] = 1.0 - w0

    oh = (jnp.logical_or(iota == e0, iota == e1)).astype(jnp.float32)
    oh16 = jnp.concatenate([oh, jnp.zeros((TOKENS, 16 - NUM_EXPERTS), jnp.float32)], axis=1)
    wi = lax.broadcasted_iota(jnp.int32, (NW, TOKENS), 0)
    ti = lax.broadcasted_iota(jnp.int32, (NW, TOKENS), 1) // TPW
    seg = (wi == ti).astype(jnp.float32)
    cnt = jnp.dot(seg, oh16, precision=_HI)
    lw = lax.broadcasted_iota(jnp.int32, (NW, NW), 0)
    lwp = lax.broadcasted_iota(jnp.int32, (NW, NW), 1)
    pre = jnp.dot((lwp < lw).astype(jnp.float32), cnt, precision=_HI)
    tot = jnp.sum(cnt, axis=0, keepdims=True)
    le = lax.broadcasted_iota(jnp.int32, (16, 16), 0)
    lep = lax.broadcasted_iota(jnp.int32, (16, 16), 1)
    offs = jnp.dot(tot, (le < lep).astype(jnp.float32), precision=_HI)
    carry_ref[...] = (offs + pre).astype(jnp.int32)
    offs_ref[...] = offs.astype(jnp.int32)


def _route(router_logits):
    return pl.pallas_call(
        _route_body,
        out_shape=(
            jax.ShapeDtypeStruct((TOKENS, 1), jnp.int32),
            jax.ShapeDtypeStruct((TOKENS, 1), jnp.int32),
            jax.ShapeDtypeStruct((TOKENS, 1), jnp.float32),
            jax.ShapeDtypeStruct((TOKENS, 1), jnp.float32),
            jax.ShapeDtypeStruct((NW, 16), jnp.int32),
            jax.ShapeDtypeStruct((1, 16), jnp.int32),
        ),
    )(router_logits)


def _scatter_body(x_hbm, a0_hbm, a1_hbm, carry_hbm, offs_hbm,
                  xs_hbm, p0_hbm, p1_hbm, bg_hbm, br_hbm,
                  a0_v, a1_v, p0_v, p1_v, carr_v, offs_v, mv_v, xt_v,
                  nxt_s, meta_s, sem, semx):
    w = _wid()
    base = w * TPW
    cx = pltpu.async_copy(x_hbm.at[pl.ds(base, TPW)], xt_v, semx)
    pltpu.sync_copy(carry_hbm.at[w], carr_v)
    pltpu.sync_copy(a0_hbm.at[pl.ds(base, TPW)], a0_v)
    pltpu.sync_copy(a1_hbm.at[pl.ds(base, TPW)], a1_v)
    carr = carr_v[...]
    for e in range(NUM_EXPERTS):
        nxt_s[e] = carr[e]
    lanes = lax.iota(jnp.int32, 16)

    @pl.when(w == 0)
    def _():
        pltpu.sync_copy(offs_hbm, offs_v)
        ov = offs_v[...]
        for e in range(NUM_EXPERTS + 1):
            meta_s[e] = ov[e]
        meta_s[16] = 0
        for sl2 in range(NBP):
            meta_s[17 + sl2] = 0
            meta_s[17 + NBP + sl2] = 0
        for m in range(ROWS // BM):
            for e in range(NUM_EXPERTS):
                hit = jnp.logical_and(meta_s[e] < (m + 1) * BM,
                                      meta_s[e + 1] > m * BM)

                @pl.when(hit)
                def _(m=m, e=e):
                    sl = meta_s[16]
                    meta_s[17 + sl] = e
                    meta_s[17 + NBP + sl] = m
                    meta_s[16] = sl + 1
        last = meta_s[16] - 1
        lb = meta_s[17 + last]
        lr = meta_s[17 + NBP + last]
        nv = meta_s[16]
        for h in range(NBP // 16):
            vb = jnp.zeros((16,), jnp.int32)
            vr = jnp.zeros((16,), jnp.int32)
            for l in range(16):
                sl2 = h * 16 + l
                pad = jnp.int32(sl2) >= nv
                vb = jnp.where(lanes == l,
                               jnp.where(pad, lb, meta_s[17 + sl2]), vb)
                vr = jnp.where(lanes == l,
                               jnp.where(pad, lr, meta_s[17 + NBP + sl2]), vr)
            mv_v[pl.ds(h * 16, 16)] = vb
            mv_v[pl.ds(NBP + h * 16, 16)] = vr
        pltpu.sync_copy(mv_v.at[pl.ds(0, NBP)], bg_hbm)
        pltpu.sync_copy(mv_v.at[pl.ds(NBP, NBP)], br_hbm)
    for c4 in range(TPW // 16):
        sl = pl.ds(c4 * 16, 16)
        for a_v_, p_v_ in ((a0_v, p0_v), (a1_v, p1_v)):
            ac = a_v_[sl]
            pos = jnp.zeros((16,), jnp.int32)
            for i in range(16):
                e_s = ac[i]
                p = nxt_s[e_s]
                nxt_s[e_s] = p + 1
                pos = jnp.where(lanes == i, p, pos)
            p_v_[sl] = pos
    cx.wait()
    c0 = pltpu.async_copy(xt_v, xs_hbm.at[p0_v], sem)
    c1 = pltpu.async_copy(xt_v, xs_hbm.at[p1_v], sem)
    pltpu.sync_copy(p0_v, p0_hbm.at[pl.ds(base, TPW)])
    pltpu.sync_copy(p1_v, p1_hbm.at[pl.ds(base, TPW)])
    c0.wait()
    c1.wait()


_scatter = functools.partial(
    pl.kernel,
    out_type=(
        jax.ShapeDtypeStruct((ROWS, HIDDEN), jnp.float32),
        jax.ShapeDtypeStruct((TOKENS,), jnp.int32),
        jax.ShapeDtypeStruct((TOKENS,), jnp.int32),
        jax.ShapeDtypeStruct((NBP,), jnp.int32),
        jax.ShapeDtypeStruct((NBP,), jnp.int32),
    ),
    mesh=_MESH,
    scratch_types=[
        pltpu.VMEM((TPW,), jnp.int32),
        pltpu.VMEM((TPW,), jnp.int32),
        pltpu.VMEM((TPW,), jnp.int32),
        pltpu.VMEM((TPW,), jnp.int32),
        pltpu.VMEM((16,), jnp.int32),
        pltpu.VMEM((16,), jnp.int32),
        pltpu.VMEM((2 * NBP,), jnp.int32),
        pltpu.VMEM((TPW, HIDDEN), jnp.float32),
        pltpu.SMEM((16,), jnp.int32),
        pltpu.SMEM((96,), jnp.int32),
        pltpu.SemaphoreType.DMA,
        pltpu.SemaphoreType.DMA,
    ],
)(_scatter_body)


def _gmm_body(bg_ref, br_ref, offs_ref, xs_ref, gw_ref, uw_ref, dw_ref, out_ref):
    i = pl.program_id(0)
    e = bg_ref[i]
    m = br_ref[i]
    base = m * BM
    ip = jnp.maximum(i - 1, 0)
    prev_m = jnp.where(i > 0, br_ref[ip], -1)
    prev_e = jnp.where(i > 0, bg_ref[ip], -1)
    dup = jnp.logical_and(m == prev_m, e == prev_e)
    first = m != prev_m
    lo = jnp.maximum(offs_ref[e] - base, 0)
    hi = jnp.minimum(offs_ref[e + 1] - base, BM)
    rows = jax.lax.broadcasted_iota(jnp.int32, (BM, 1), 0)

    @pl.when(first)
    def _():
        out_ref[...] = jnp.zeros_like(out_ref)

    @pl.when(jnp.logical_not(dup))
    def _():
        x = xs_ref[...].astype(jnp.bfloat16)
        gw = gw_ref[0].astype(jnp.bfloat16)
        uw = uw_ref[0].astype(jnp.bfloat16)
        dw = dw_ref[0].astype(jnp.bfloat16)
        g = jnp.dot(x, gw, preferred_element_type=jnp.float32)
        u = jnp.dot(x, uw, preferred_element_type=jnp.float32)
        h = ((g * jax.nn.sigmoid(g)) * u).astype(jnp.bfloat16)
        y = jnp.dot(h, dw, preferred_element_type=jnp.float32)
        msk = jnp.logical_and(rows >= lo, rows < hi)
        out_ref[...] += jnp.where(msk, y, 0.0)


def _gmm(bg, br, offs, xs, gw, uw, dw):
    return pl.pallas_call(
        _gmm_body,
        grid_spec=pltpu.PrefetchScalarGridSpec(
            num_scalar_prefetch=3,
            grid=(NB,),
            in_specs=[
                pl.BlockSpec((BM, HIDDEN), lambda i, bg, br, offs: (br[i], 0)),
                pl.BlockSpec((1, HIDDEN, INTERMEDIATE),
                             lambda i, bg, br, offs: (bg[i], 0, 0)),
                pl.BlockSpec((1, HIDDEN, INTERMEDIATE),
                             lambda i, bg, br, offs: (bg[i], 0, 0)),
                pl.BlockSpec((1, INTERMEDIATE, HIDDEN),
                             lambda i, bg, br, offs: (bg[i], 0, 0)),
            ],
            out_specs=pl.BlockSpec((BM, HIDDEN), lambda i, bg, br, offs: (br[i], 0)),
        ),
        out_shape=jax.ShapeDtypeStruct((ROWS, HIDDEN), jnp.float32),
    )(bg, br, offs, xs, gw, uw, dw)


def _combine_body(y_hbm, p0_hbm, p1_hbm, w0_hbm, w1_hbm, out_hbm,
                  p0_v, p1_v, w0_v, w1_v, r0_v, r1_v, o_v, sem):
    w = _wid()
    base = w * TPW
    pltpu.sync_copy(p0_hbm.at[pl.ds(base, TPW)], p0_v)
    pltpu.sync_copy(p1_hbm.at[pl.ds(base, TPW)], p1_v)
    pltpu.sync_copy(w0_hbm.at[pl.ds(base, TPW)], w0_v)
    pltpu.sync_copy(w1_hbm.at[pl.ds(base, TPW)], w1_v)
    HT = TPW // 2
    for hh in range(2):
        g0 = pltpu.async_copy(y_hbm.at[p0_v.at[pl.ds(hh * HT, HT)]], r0_v, sem)
        g1 = pltpu.async_copy(y_hbm.at[p1_v.at[pl.ds(hh * HT, HT)]], r1_v, sem)
        g0.wait()
        g1.wait()
        for g in range(HT // 16):
            w0g = w0_v[pl.ds(hh * HT + g * 16, 16)]
            w1g = w1_v[pl.ds(hh * HT + g * 16, 16)]

            def body(f, _, g=g, w0g=w0g, w1g=w1g):
                sl = pl.ds(f * 16, 16)
                for i in range(16):
                    r = g * 16 + i
                    o_v[r, sl] = w0g[i] * r0_v[r, sl] + w1g[i] * r1_v[r, sl]
                return 0

            lax.fori_loop(0, HIDDEN // 16, body, 0)
        pltpu.sync_copy(o_v, out_hbm.at[pl.ds(base + hh * HT, HT)])


_combine = functools.partial(
    pl.kernel,
    out_type=jax.ShapeDtypeStruct((TOKENS, HIDDEN), jnp.float32),
    mesh=_MESH,
    scratch_types=[
        pltpu.VMEM((TPW,), jnp.int32),
        pltpu.VMEM((TPW,), jnp.int32),
        pltpu.VMEM((TPW,), jnp.float32),
        pltpu.VMEM((TPW,), jnp.float32),
        pltpu.VMEM((TPW // 2, HIDDEN), jnp.float32),
        pltpu.VMEM((TPW // 2, HIDDEN), jnp.float32),
        pltpu.VMEM((TPW // 2, HIDDEN), jnp.float32),
        pltpu.SemaphoreType.DMA,
    ],
)(_combine_body)


def kernel(hidden_states, router_logits, gate_w, up_w, down_w):
    a0, a1, w0, w1, carry, offs = _route(router_logits)
    offs_flat = offs.reshape(16)
    xs, p0, p1, bg, br = _scatter(hidden_states, a0.reshape(TOKENS),
                                  a1.reshape(TOKENS), carry, offs_flat)
    ys = _gmm(bg, br, offs_flat, xs, gate_w, up_w, down_w)
    return _combine(ys, p0, p1, w0.reshape(TOKENS), w1.reshape(TOKENS))


# speedup vs baseline: 1.1703x; 1.0233x over previous
import functools

import jax
import jax.numpy as jnp
from jax import lax
from jax.experimental import pallas as pl
from jax.experimental.pallas import tpu as pltpu
from jax.experimental.pallas import tpu_sc as plsc

NUM_EXPERTS = 8
TOP_K = 2
HIDDEN = 768
INTERMEDIATE = 512
TOKENS = 2048
ROWS = TOKENS * TOP_K

NC = 2
NS = 16
NW = NC * NS
TPW = TOKENS // NW
BM = 512
NB = ROWS // BM + NUM_EXPERTS - 1
NBP = 16

_MESH = plsc.VectorSubcoreMesh(core_axis_name="c", subcore_axis_name="s")
_HI = jax.lax.Precision.HIGHEST


def _wid():
    return lax.axis_index("c") * NS + lax.axis_index("s")


def _route_body(lg_ref, a0_ref, a1_ref, w0_ref, w1_ref, carry_ref, offs_ref):
    logits = lg_ref[...]
    iota = lax.broadcasted_iota(jnp.int32, (TOKENS, NUM_EXPERTS), 1)
    big = jnp.int32(NUM_EXPERTS)
    m1 = jnp.max(logits, axis=1, keepdims=True)
    e0 = jnp.min(jnp.where(logits == m1, iota, big), axis=1, keepdims=True)
    masked = jnp.where(iota == e0, -jnp.inf, logits)
    m2 = jnp.max(masked, axis=1, keepdims=True)
    e1 = jnp.min(jnp.where(masked == m2, iota, big), axis=1, keepdims=True)
    w0 = 1.0 / (1.0 + jnp.exp(m2 - m1))
    a0_ref[...] = e0.reshape(TOKENS)
    a1_ref[...] = e1.reshape(TOKENS)
    w0_ref[...] = w0.reshape(TOKENS)
    w1_ref[...] = (1.0 - w0).reshape(TOKENS)

    oh = (jnp.logical_or(iota == e0, iota == e1)).astype(jnp.float32)
    oh16 = jnp.concatenate([oh, jnp.zeros((TOKENS, 16 - NUM_EXPERTS), jnp.float32)], axis=1)
    wi = lax.broadcasted_iota(jnp.int32, (NW, TOKENS), 0)
    ti = lax.broadcasted_iota(jnp.int32, (NW, TOKENS), 1) // TPW
    seg = (wi == ti).astype(jnp.float32)
    cnt = jnp.dot(seg, oh16, precision=_HI)
    lw = lax.broadcasted_iota(jnp.int32, (NW, NW), 0)
    lwp = lax.broadcasted_iota(jnp.int32, (NW, NW), 1)
    pre = jnp.dot((lwp < lw).astype(jnp.float32), cnt, precision=_HI)
    tot = jnp.sum(cnt, axis=0, keepdims=True)
    le = lax.broadcasted_iota(jnp.int32, (16, 16), 0)
    lep = lax.broadcasted_iota(jnp.int32, (16, 16), 1)
    offs = jnp.dot(tot, (le < lep).astype(jnp.float32), precision=_HI)
    carry_ref[...] = (offs + pre).astype(jnp.int32)
    offs_ref[...] = offs.astype(jnp.int32)


def _route(router_logits):
    return pl.pallas_call(
        _route_body,
        out_shape=(
            jax.ShapeDtypeStruct((TOKENS,), jnp.int32),
            jax.ShapeDtypeStruct((TOKENS,), jnp.int32),
            jax.ShapeDtypeStruct((TOKENS,), jnp.float32),
            jax.ShapeDtypeStruct((TOKENS,), jnp.float32),
            jax.ShapeDtypeStruct((NW, 16), jnp.int32),
            jax.ShapeDtypeStruct((1, 16), jnp.int32),
        ),
    )(router_logits)


def _scatter_body(x_hbm, a0_hbm, a1_hbm, carry_hbm, offs_hbm,
                  xs_hbm, p0_hbm, p1_hbm, bg_hbm, br_hbm,
                  a0_v, a1_v, p0_v, p1_v, carr_v, offs_v, mv_v, xt_v,
                  nxt_s, meta_s, sem, semx):
    w = _wid()
    base = w * TPW
    cx = pltpu.async_copy(x_hbm.at[pl.ds(base, TPW)], xt_v, semx)
    pltpu.sync_copy(carry_hbm.at[w], carr_v)
    pltpu.sync_copy(a0_hbm.at[pl.ds(base, TPW)], a0_v)
    pltpu.sync_copy(a1_hbm.at[pl.ds(base, TPW)], a1_v)
    carr = carr_v[...]
    for e in range(NUM_EXPERTS):
        nxt_s[e] = carr[e]
    lanes = lax.iota(jnp.int32, 16)

    @pl.when(w == 0)
    def _():
        pltpu.sync_copy(offs_hbm, offs_v)
        ov = offs_v[...]
        for e in range(NUM_EXPERTS + 1):
            meta_s[e] = ov[e]
        meta_s[16] = 0
        for sl2 in range(NBP):
            meta_s[17 + sl2] = 0
            meta_s[17 + NBP + sl2] = 0
        for m in range(ROWS // BM):
            for e in range(NUM_EXPERTS):
                hit = jnp.logical_and(meta_s[e] < (m + 1) * BM,
                                      meta_s[e + 1] > m * BM)

                @pl.when(hit)
                def _(m=m, e=e):
                    sl = meta_s[16]
                    meta_s[17 + sl] = e
                    meta_s[17 + NBP + sl] = m
                    meta_s[16] = sl + 1
        last = meta_s[16] - 1
        lb = meta_s[17 + last]
        lr = meta_s[17 + NBP + last]
        nv = meta_s[16]
        for h in range(NBP // 16):
            vb = jnp.zeros((16,), jnp.int32)
            vr = jnp.zeros((16,), jnp.int32)
            for l in range(16):
                sl2 = h * 16 + l
                pad = jnp.int32(sl2) >= nv
                vb = jnp.where(lanes == l,
                               jnp.where(pad, lb, meta_s[17 + sl2]), vb)
                vr = jnp.where(lanes == l,
                               jnp.where(pad, lr, meta_s[17 + NBP + sl2]), vr)
            mv_v[pl.ds(h * 16, 16)] = vb
            mv_v[pl.ds(NBP + h * 16, 16)] = vr
        pltpu.sync_copy(mv_v.at[pl.ds(0, NBP)], bg_hbm)
        pltpu.sync_copy(mv_v.at[pl.ds(NBP, NBP)], br_hbm)
    for c4 in range(TPW // 16):
        sl = pl.ds(c4 * 16, 16)
        for a_v_, p_v_ in ((a0_v, p0_v), (a1_v, p1_v)):
            ac = a_v_[sl]
            pos = jnp.zeros((16,), jnp.int32)
            for i in range(16):
                e_s = ac[i]
                p = nxt_s[e_s]
                nxt_s[e_s] = p + 1
                pos = jnp.where(lanes == i, p, pos)
            p_v_[sl] = pos
    cx.wait()
    c0 = pltpu.async_copy(xt_v, xs_hbm.at[p0_v], sem)
    c1 = pltpu.async_copy(xt_v, xs_hbm.at[p1_v], sem)
    pltpu.sync_copy(p0_v, p0_hbm.at[pl.ds(base, TPW)])
    pltpu.sync_copy(p1_v, p1_hbm.at[pl.ds(base, TPW)])
    c0.wait()
    c1.wait()


_scatter = functools.partial(
    pl.kernel,
    out_type=(
        jax.ShapeDtypeStruct((ROWS, HIDDEN), jnp.float32),
        jax.ShapeDtypeStruct((TOKENS,), jnp.int32),
        jax.ShapeDtypeStruct((TOKENS,), jnp.int32),
        jax.ShapeDtypeStruct((NBP,), jnp.int32),
        jax.ShapeDtypeStruct((NBP,), jnp.int32),
    ),
    mesh=_MESH,
    scratch_types=[
        pltpu.VMEM((TPW,), jnp.int32),
        pltpu.VMEM((TPW,), jnp.int32),
        pltpu.VMEM((TPW,), jnp.int32),
        pltpu.VMEM((TPW,), jnp.int32),
        pltpu.VMEM((16,), jnp.int32),
        pltpu.VMEM((16,), jnp.int32),
        pltpu.VMEM((2 * NBP,), jnp.int32),
        pltpu.VMEM((TPW, HIDDEN), jnp.float32),
        pltpu.SMEM((16,), jnp.int32),
        pltpu.SMEM((96,), jnp.int32),
        pltpu.SemaphoreType.DMA,
        pltpu.SemaphoreType.DMA,
    ],
)(_scatter_body)


def _gmm_body(bg_ref, br_ref, offs_ref, xs_ref, gw_ref, uw_ref, dw_ref, out_ref):
    i = pl.program_id(0)
    e = bg_ref[i]
    m = br_ref[i]
    base = m * BM
    ip = jnp.maximum(i - 1, 0)
    prev_m = jnp.where(i > 0, br_ref[ip], -1)
    prev_e = jnp.where(i > 0, bg_ref[ip], -1)
    dup = jnp.logical_and(m == prev_m, e == prev_e)
    first = m != prev_m
    lo = jnp.maximum(offs_ref[e] - base, 0)
    hi = jnp.minimum(offs_ref[e + 1] - base, BM)
    rows = jax.lax.broadcasted_iota(jnp.int32, (BM, 1), 0)

    @pl.when(first)
    def _():
        out_ref[...] = jnp.zeros_like(out_ref)

    @pl.when(jnp.logical_not(dup))
    def _():
        x = xs_ref[...].astype(jnp.bfloat16)
        gw = gw_ref[0].astype(jnp.bfloat16)
        uw = uw_ref[0].astype(jnp.bfloat16)
        dw = dw_ref[0].astype(jnp.bfloat16)
        g = jnp.dot(x, gw, preferred_element_type=jnp.float32)
        u = jnp.dot(x, uw, preferred_element_type=jnp.float32)
        h = ((g * jax.nn.sigmoid(g)) * u).astype(jnp.bfloat16)
        y = jnp.dot(h, dw, preferred_element_type=jnp.float32)
        msk = jnp.logical_and(rows >= lo, rows < hi)
        out_ref[...] += jnp.where(msk, y, 0.0)


def _gmm(bg, br, offs, xs, gw, uw, dw):
    return pl.pallas_call(
        _gmm_body,
        grid_spec=pltpu.PrefetchScalarGridSpec(
            num_scalar_prefetch=3,
            grid=(NB,),
            in_specs=[
                pl.BlockSpec((BM, HIDDEN), lambda i, bg, br, offs: (br[i], 0)),
                pl.BlockSpec((1, HIDDEN, INTERMEDIATE),
                             lambda i, bg, br, offs: (bg[i], 0, 0)),
                pl.BlockSpec((1, HIDDEN, INTERMEDIATE),
                             lambda i, bg, br, offs: (bg[i], 0, 0)),
                pl.BlockSpec((1, INTERMEDIATE, HIDDEN),
                             lambda i, bg, br, offs: (bg[i], 0, 0)),
            ],
            out_specs=pl.BlockSpec((BM, HIDDEN), lambda i, bg, br, offs: (br[i], 0)),
        ),
        out_shape=jax.ShapeDtypeStruct((ROWS, HIDDEN), jnp.float32),
    )(bg, br, offs, xs, gw, uw, dw)


def _combine_body(y_hbm, p0_hbm, p1_hbm, w0_hbm, w1_hbm, out_hbm,
                  p0_v, p1_v, w0_v, w1_v, r0_v, r1_v, o_v, sem):
    w = _wid()
    base = w * TPW
    pltpu.sync_copy(p0_hbm.at[pl.ds(base, TPW)], p0_v)
    pltpu.sync_copy(p1_hbm.at[pl.ds(base, TPW)], p1_v)
    pltpu.sync_copy(w0_hbm.at[pl.ds(base, TPW)], w0_v)
    pltpu.sync_copy(w1_hbm.at[pl.ds(base, TPW)], w1_v)
    HT = TPW // 2
    for hh in range(2):
        g0 = pltpu.async_copy(y_hbm.at[p0_v.at[pl.ds(hh * HT, HT)]], r0_v, sem)
        g1 = pltpu.async_copy(y_hbm.at[p1_v.at[pl.ds(hh * HT, HT)]], r1_v, sem)
        g0.wait()
        g1.wait()
        for g in range(HT // 16):
            w0g = w0_v[pl.ds(hh * HT + g * 16, 16)]
            w1g = w1_v[pl.ds(hh * HT + g * 16, 16)]

            def body(f, _, g=g, w0g=w0g, w1g=w1g):
                sl = pl.ds(f * 16, 16)
                for i in range(16):
                    r = g * 16 + i
                    o_v[r, sl] = w0g[i] * r0_v[r, sl] + w1g[i] * r1_v[r, sl]
                return 0

            lax.fori_loop(0, HIDDEN // 16, body, 0)
        pltpu.sync_copy(o_v, out_hbm.at[pl.ds(base + hh * HT, HT)])


_combine = functools.partial(
    pl.kernel,
    out_type=jax.ShapeDtypeStruct((TOKENS, HIDDEN), jnp.float32),
    mesh=_MESH,
    scratch_types=[
        pltpu.VMEM((TPW,), jnp.int32),
        pltpu.VMEM((TPW,), jnp.int32),
        pltpu.VMEM((TPW,), jnp.float32),
        pltpu.VMEM((TPW,), jnp.float32),
        pltpu.VMEM((TPW // 2, HIDDEN), jnp.float32),
        pltpu.VMEM((TPW // 2, HIDDEN), jnp.float32),
        pltpu.VMEM((TPW // 2, HIDDEN), jnp.float32),
        pltpu.SemaphoreType.DMA,
    ],
)(_combine_body)


def kernel(hidden_states, router_logits, gate_w, up_w, down_w):
    a0, a1, w0, w1, carry, offs = _route(router_logits)
    offs_flat = offs.reshape(16)
    xs, p0, p1, bg, br = _scatter(hidden_states, a0, a1, carry, offs_flat)
    ys = _gmm(bg, br, offs_flat, xs, gate_w, up_w, down_w)
    return _combine(ys, p0, p1, w0, w1)
